# baseline (speedup 1.0000x reference)
import functools

import jax
import jax.numpy as jnp
from jax import lax
from jax.experimental import pallas as pl
from jax.experimental.pallas import tpu as pltpu
from jax.experimental.pallas import tpu_sc as plsc

N = 10000
E = 320000
D = 128
HD = D // 2

NC = 2
NS = 16
CHUNK = 128
ROWS_PER_W = 160
EPAD = NS * ROWS_PER_W * CHUNK
NPAD = EPAD - E
PSRC = 128
SLICE = 624
TAIL = N - NS * SLICE
RB = 1000


def _project_body(x_ref, w_ref, al_ref, ar_ref, feat_ref, el_ref, er_ref):
    f = jnp.dot(x_ref[...], w_ref[...], preferred_element_type=jnp.float32)
    feat_ref[0] = f[:, :HD]
    feat_ref[1] = f[:, HD:]
    el_ref[...] = jnp.sum(f * al_ref[...], axis=1, keepdims=True)
    er_ref[...] = jnp.sum(f * ar_ref[...], axis=1, keepdims=True)


def _project(x, W, attn_l, attn_r):
    grid = (N // RB,)
    return pl.pallas_call(
        _project_body,
        grid=grid,
        in_specs=[
            pl.BlockSpec((RB, D), lambda i: (i, 0)),
            pl.BlockSpec((D, D), lambda i: (0, 0)),
            pl.BlockSpec((1, D), lambda i: (0, 0)),
            pl.BlockSpec((1, D), lambda i: (0, 0)),
        ],
        out_specs=[
            pl.BlockSpec((NC, RB, HD), lambda i: (0, i, 0)),
            pl.BlockSpec((RB, 1), lambda i: (i, 0)),
            pl.BlockSpec((RB, 1), lambda i: (i, 0)),
        ],
        out_shape=[
            jax.ShapeDtypeStruct((NC, N, HD), jnp.float32),
            jax.ShapeDtypeStruct((N, 1), jnp.float32),
            jax.ShapeDtypeStruct((N, 1), jnp.float32),
        ],
    )(x, W, attn_l, attn_r)


def _sc_body(feat2_hbm, eler_hbm, sd_hbm,
             numer_hbm, denom_hbm,
             sd_v, rows_v, rows_v2, zb_v,
             srcr_v, srcc_v, dstc_v, elc_v, erc_v, eec_v,
             srcr_v2, srcc_v2, dstc_v2, elc_v2, erc_v2, eec_v2,
             feat_sp, el_sp, er_sp, acc, accd,
             sem, seme, sems, sem2, seme2, sems2):
    cid = lax.axis_index("c")
    sid = lax.axis_index("s")
    zero16 = jnp.zeros((16,), jnp.float32)
    nmax16 = jnp.full((16,), N - 1, jnp.int32)

    def _zrow(i, _):
        for c in range(HD // 16):
            rows_v[i, pl.ds(16 * c, 16)] = zero16
        return _
    lax.fori_loop(0, CHUNK, _zrow, 0, unroll=4)

    def _zb(i, _):
        zb_v[pl.ds(16 * i, 16)] = zero16
        return _
    lax.fori_loop(0, SLICE // 16, _zb, 0, unroll=4)

    def _zacc(k, _):
        pltpu.sync_copy(rows_v.at[pl.ds(0, 104)],
                        acc.at[pl.ds(sid * SLICE + k * 104, 104)])
        return _
    lax.fori_loop(0, SLICE // 104, _zacc, 0)
    pltpu.sync_copy(zb_v, accd.at[pl.ds(sid * SLICE, SLICE)])

    @pl.when(sid == 0)
    def _zero_tail():
        pltpu.sync_copy(rows_v.at[pl.ds(0, TAIL)],
                        acc.at[pl.ds(NS * SLICE, TAIL)])
        pltpu.sync_copy(zb_v.at[pl.ds(0, TAIL)],
                        accd.at[pl.ds(NS * SLICE, TAIL)])

    pltpu.sync_copy(feat2_hbm.at[cid, pl.ds(sid * SLICE, SLICE)],
                    feat_sp.at[pl.ds(sid * SLICE, SLICE)])

    @pl.when(sid == 0)
    def _stage_tail():
        pltpu.sync_copy(feat2_hbm.at[cid, pl.ds(NS * SLICE, TAIL)],
                        feat_sp.at[pl.ds(NS * SLICE, TAIL)])
        pltpu.sync_copy(eler_hbm.at[0], el_sp)

    @pl.when(sid == 1)
    def _stage_er():
        pltpu.sync_copy(eler_hbm.at[1, pl.ds(0, N)], er_sp)

    base = sid * ROWS_PER_W
    pltpu.sync_copy(sd_hbm.at[pl.ds(base, ROWS_PER_W)], sd_v)

    plsc.subcore_barrier()

    mask16 = jnp.full((16,), (1 << 14) - 1, jnp.int32)
    sh14 = jnp.full((16,), 14, jnp.int32)
    lane_splat = [jnp.full((16,), l, jnp.int32) for l in range(16)]

    bufs = ((rows_v, srcr_v, srcc_v, dstc_v, elc_v, erc_v, eec_v,
             sem, seme, sems),
            (rows_v2, srcr_v2, srcc_v2, dstc_v2, elc_v2, erc_v2, eec_v2,
             sem2, seme2, sems2))

    def _start(j, rows, srcr, srcc, dstc, elc, erc, eec, s, se, ss):
        for c in range(CHUNK // 16):
            p16 = sd_v[j, pl.ds(16 * c, 16)]
            s16 = p16 & mask16
            srcr[pl.ds(16 * c, 16)] = s16
            srcc[pl.ds(16 * c, 16)] = jnp.minimum(s16, nmax16)
            dstc[pl.ds(16 * c, 16)] = lax.shift_right_logical(p16, sh14)
        pltpu.async_copy(feat_sp.at[srcc], rows, s)
        pltpu.async_copy(el_sp.at[srcr], elc, se)
        pltpu.async_copy(er_sp.at[dstc], erc, se)

    def _finish(j, rows, srcr, srcc, dstc, elc, erc, eec, s, se, ss):
        pltpu.make_async_copy(el_sp.at[srcr], elc, se).wait()
        pltpu.make_async_copy(er_sp.at[dstc], erc, se).wait()
        pltpu.make_async_copy(feat_sp.at[srcc], rows, s).wait()

        for g in range(CHUNK // 16):
            e = elc[pl.ds(16 * g, 16)] + erc[pl.ds(16 * g, 16)]
            e = jnp.maximum(e, e * 0.2)
            a16 = jnp.exp(e)
            eec[pl.ds(16 * g, 16)] = a16
            for l in range(16):
                al = a16[lane_splat[l]]
                i = 16 * g + l
                for c in range(HD // 16):
                    rows[i, pl.ds(16 * c, 16)] = (
                        rows[i, pl.ds(16 * c, 16)] * al)

        pltpu.async_copy(rows, acc.at[dstc], ss, add=True)
        pltpu.async_copy(eec, accd.at[dstc], ss, add=True)

    def _wait_scatter(rows, srcr, srcc, dstc, elc, erc, eec, s, se, ss):
        pltpu.make_async_copy(rows, acc.at[dstc], ss).wait()
        pltpu.make_async_copy(eec, accd.at[dstc], ss).wait()

    _start(0, *bufs[0])
    _start(1, *bufs[1])

    def _pair(jj, _):
        j0 = 2 * jj
        _finish(j0, *bufs[0])
        _finish(j0 + 1, *bufs[1])

        @pl.when(j0 + 2 < ROWS_PER_W)
        def _next0():
            _wait_scatter(*bufs[0])
            _start(j0 + 2, *bufs[0])

        @pl.when(j0 + 3 < ROWS_PER_W)
        def _next1():
            _wait_scatter(*bufs[1])
            _start(j0 + 3, *bufs[1])
        return _
    lax.fori_loop(0, ROWS_PER_W // 2, _pair, 0)

    _wait_scatter(*bufs[0])
    _wait_scatter(*bufs[1])
    plsc.subcore_barrier()

    pltpu.sync_copy(acc.at[pl.ds(sid * SLICE, SLICE)],
                    numer_hbm.at[cid, pl.ds(sid * SLICE, SLICE)])
    pltpu.sync_copy(accd.at[pl.ds(sid * SLICE, SLICE)],
                    denom_hbm.at[cid, pl.ds(sid * SLICE, SLICE)])

    @pl.when(sid == 0)
    def _dump_tail():
        pltpu.sync_copy(acc.at[pl.ds(NS * SLICE, TAIL)],
                        numer_hbm.at[cid, pl.ds(NS * SLICE, TAIL)])
        pltpu.sync_copy(accd.at[pl.ds(NS * SLICE, TAIL)],
                        denom_hbm.at[cid, pl.ds(NS * SLICE, TAIL)])


def _sc_edges(feat2, eler, sd):
    mesh = plsc.VectorSubcoreMesh(core_axis_name="c", subcore_axis_name="s")
    chunk_bufs = [
        pltpu.VMEM((CHUNK, HD), jnp.float32),
        pltpu.VMEM((CHUNK, HD), jnp.float32),
        pltpu.VMEM((SLICE,), jnp.float32),
    ]
    per_set = [
        pltpu.VMEM((CHUNK,), jnp.int32),
        pltpu.VMEM((CHUNK,), jnp.int32),
        pltpu.VMEM((CHUNK,), jnp.int32),
        pltpu.VMEM((CHUNK,), jnp.float32),
        pltpu.VMEM((CHUNK,), jnp.float32),
        pltpu.VMEM((CHUNK,), jnp.float32),
    ]
    fn = functools.partial(
        pl.kernel,
        out_type=(
            jax.ShapeDtypeStruct((NC, N, HD), jnp.float32),
            jax.ShapeDtypeStruct((NC, N), jnp.float32),
        ),
        mesh=mesh,
        compiler_params=pltpu.CompilerParams(
            needs_layout_passes=False, use_tc_tiling_on_sc=False),
        scratch_types=[
            pltpu.VMEM((ROWS_PER_W, CHUNK), jnp.int32),
        ] + chunk_bufs + per_set + per_set + [
            pltpu.VMEM_SHARED((N, HD), jnp.float32),
            pltpu.VMEM_SHARED((N + PSRC,), jnp.float32),
            pltpu.VMEM_SHARED((N,), jnp.float32),
            pltpu.VMEM_SHARED((N, HD), jnp.float32),
            pltpu.VMEM_SHARED((N,), jnp.float32),
            pltpu.SemaphoreType.DMA,
            pltpu.SemaphoreType.DMA,
            pltpu.SemaphoreType.DMA,
            pltpu.SemaphoreType.DMA,
            pltpu.SemaphoreType.DMA,
            pltpu.SemaphoreType.DMA,
        ],
    )(_sc_body)
    return fn(feat2, eler, sd)


def _combine_body(n0_ref, n1_ref, d_ref, b_ref, out_ref):
    den = jnp.maximum(d_ref[...], 1e-9)
    out_ref[:, :HD] = n0_ref[...] / den + b_ref[:, :HD]
    out_ref[:, HD:] = n1_ref[...] / den + b_ref[:, HD:]


def _combine(numer, denom, bias):
    n0, n1 = numer[0], numer[1]
    d = denom[0].reshape(N, 1)
    b = bias.reshape(1, D)
    grid = (N // RB,)
    return pl.pallas_call(
        _combine_body,
        grid=grid,
        in_specs=[
            pl.BlockSpec((RB, HD), lambda i: (i, 0)),
            pl.BlockSpec((RB, HD), lambda i: (i, 0)),
            pl.BlockSpec((RB, 1), lambda i: (i, 0)),
            pl.BlockSpec((1, D), lambda i: (0, 0)),
        ],
        out_specs=pl.BlockSpec((RB, D), lambda i: (i, 0)),
        out_shape=jax.ShapeDtypeStruct((N, D), jnp.float32),
    )(n0, n1, d, b)


def kernel(x, edge_index, W, attn_l, attn_r, bias):
    src = edge_index[0]
    dst = edge_index[1]
    pad_src = N + (jnp.arange(NPAD, dtype=jnp.int32) % PSRC)
    pad_dst = (jnp.arange(NPAD, dtype=jnp.int32) * 13) % N
    src_p = jnp.concatenate([src, pad_src])
    dst_p = jnp.concatenate([dst, pad_dst])
    sd = (src_p | (dst_p << 14)).reshape(EPAD // CHUNK, CHUNK)

    feat2, el, er = _project(x, W, attn_l, attn_r)
    eler = jnp.stack([
        jnp.concatenate([el.reshape(N), jnp.full((PSRC,), -1e30, jnp.float32)]),
        jnp.concatenate([er.reshape(N), jnp.zeros((PSRC,), jnp.float32)]),
    ])
    numer, denom = _sc_edges(feat2, eler, sd)
    out = _combine(numer, denom, bias)
    return out.reshape(N, 1, D)


# speedup vs baseline: 1.1057x; 1.1057x over previous
import functools

import jax
import jax.numpy as jnp
from jax import lax
from jax.experimental import pallas as pl
from jax.experimental.pallas import tpu as pltpu
from jax.experimental.pallas import tpu_sc as plsc

N = 10000
E = 320000
D = 128
HD = D // 2

NC = 2
NS = 16
CHUNK = 128
ROWS_PER_W = 160
EPAD = NS * ROWS_PER_W * CHUNK
NPAD = EPAD - E
PSRC = 128
SLICE = 624
TAIL = N - NS * SLICE
RB = 1000


def _project_body(x_ref, w_ref, al_ref, ar_ref, feat_ref, el_ref, er_ref):
    f = jnp.dot(x_ref[...], w_ref[...], preferred_element_type=jnp.float32)
    feat_ref[0] = f[:, :HD]
    feat_ref[1] = f[:, HD:]
    el_ref[...] = jnp.sum(f * al_ref[...], axis=1, keepdims=True)
    er_ref[...] = jnp.sum(f * ar_ref[...], axis=1, keepdims=True)


def _project(x, W, attn_l, attn_r):
    grid = (N // RB,)
    return pl.pallas_call(
        _project_body,
        grid=grid,
        in_specs=[
            pl.BlockSpec((RB, D), lambda i: (i, 0)),
            pl.BlockSpec((D, D), lambda i: (0, 0)),
            pl.BlockSpec((1, D), lambda i: (0, 0)),
            pl.BlockSpec((1, D), lambda i: (0, 0)),
        ],
        out_specs=[
            pl.BlockSpec((NC, RB, HD), lambda i: (0, i, 0)),
            pl.BlockSpec((RB, 1), lambda i: (i, 0)),
            pl.BlockSpec((RB, 1), lambda i: (i, 0)),
        ],
        out_shape=[
            jax.ShapeDtypeStruct((NC, N, HD), jnp.float32),
            jax.ShapeDtypeStruct((N, 1), jnp.float32),
            jax.ShapeDtypeStruct((N, 1), jnp.float32),
        ],
    )(x, W, attn_l, attn_r)


def _sc_body(feat2_hbm, eler_hbm, sd_hbm,
             numer_hbm, denom_hbm,
             sd_v, rows_v, rows_v2, zb_v,
             srcr_v, srcc_v, dstc_v, elc_v, erc_v, eec_v,
             srcr_v2, srcc_v2, dstc_v2, elc_v2, erc_v2, eec_v2,
             feat_sp, el_sp, er_sp, acc, accd,
             sem, seme, sems, sem2, seme2, sems2):
    cid = lax.axis_index("c")
    sid = lax.axis_index("s")
    zero16 = jnp.zeros((16,), jnp.float32)
    nmax16 = jnp.full((16,), N - 1, jnp.int32)

    def _zrow(i, _):
        for c in range(HD // 16):
            rows_v[i, pl.ds(16 * c, 16)] = zero16
        return _
    lax.fori_loop(0, CHUNK, _zrow, 0, unroll=4)

    def _zb(i, _):
        zb_v[pl.ds(16 * i, 16)] = zero16
        return _
    lax.fori_loop(0, SLICE // 16, _zb, 0, unroll=4)

    def _zacc(k, _):
        pltpu.sync_copy(rows_v.at[pl.ds(0, 104)],
                        acc.at[pl.ds(sid * SLICE + k * 104, 104)])
        return _
    lax.fori_loop(0, SLICE // 104, _zacc, 0)
    pltpu.sync_copy(zb_v, accd.at[pl.ds(sid * SLICE, SLICE)])

    @pl.when(sid == 0)
    def _zero_tail():
        pltpu.sync_copy(rows_v.at[pl.ds(0, TAIL)],
                        acc.at[pl.ds(NS * SLICE, TAIL)])
        pltpu.sync_copy(zb_v.at[pl.ds(0, TAIL)],
                        accd.at[pl.ds(NS * SLICE, TAIL)])

    pltpu.sync_copy(feat2_hbm.at[cid, pl.ds(sid * SLICE, SLICE)],
                    feat_sp.at[pl.ds(sid * SLICE, SLICE)])

    @pl.when(sid == 0)
    def _stage_tail():
        pltpu.sync_copy(feat2_hbm.at[cid, pl.ds(NS * SLICE, TAIL)],
                        feat_sp.at[pl.ds(NS * SLICE, TAIL)])
        pltpu.sync_copy(eler_hbm.at[0], el_sp)

    @pl.when(sid == 1)
    def _stage_er():
        pltpu.sync_copy(eler_hbm.at[1, pl.ds(0, N)], er_sp)

    base = sid * ROWS_PER_W
    pltpu.sync_copy(sd_hbm.at[pl.ds(base, ROWS_PER_W)], sd_v)

    plsc.subcore_barrier()

    mask16 = jnp.full((16,), (1 << 14) - 1, jnp.int32)
    sh14 = jnp.full((16,), 14, jnp.int32)
    lane_splat = [jnp.full((16,), l, jnp.int32) for l in range(16)]

    bufs = ((rows_v, srcr_v, srcc_v, dstc_v, elc_v, erc_v, eec_v,
             sem, seme, sems),
            (rows_v2, srcr_v2, srcc_v2, dstc_v2, elc_v2, erc_v2, eec_v2,
             sem2, seme2, sems2))

    def _start(j, rows, srcr, srcc, dstc, elc, erc, eec, s, se, ss):
        for c in range(CHUNK // 16):
            p16 = sd_v[j, pl.ds(16 * c, 16)]
            s16 = p16 & mask16
            srcr[pl.ds(16 * c, 16)] = s16
            srcc[pl.ds(16 * c, 16)] = jnp.minimum(s16, nmax16)
            dstc[pl.ds(16 * c, 16)] = lax.shift_right_logical(p16, sh14)
        pltpu.async_copy(feat_sp.at[srcc], rows, s)
        pltpu.async_copy(el_sp.at[srcr], elc, se)
        pltpu.async_copy(er_sp.at[dstc], erc, se)

    def _finish(j, rows, srcr, srcc, dstc, elc, erc, eec, s, se, ss):
        pltpu.make_async_copy(el_sp.at[srcr], elc, se).wait()
        pltpu.make_async_copy(er_sp.at[dstc], erc, se).wait()
        pltpu.make_async_copy(feat_sp.at[srcc], rows, s).wait()

        for g in range(CHUNK // 16):
            e = elc[pl.ds(16 * g, 16)] + erc[pl.ds(16 * g, 16)]
            e = jnp.maximum(e, e * 0.2)
            a16 = jnp.exp(e)
            eec[pl.ds(16 * g, 16)] = a16
            for l in range(16):
                al = a16[lane_splat[l]]
                i = 16 * g + l
                for c in range(HD // 16):
                    rows[i, pl.ds(16 * c, 16)] = (
                        rows[i, pl.ds(16 * c, 16)] * al)

        pltpu.sync_copy(rows, acc.at[dstc], add=True)
        pltpu.sync_copy(eec, accd.at[dstc], add=True)

    _start(0, *bufs[0])

    def _pair(jj, _):
        j0 = 2 * jj
        _start(j0 + 1, *bufs[1])
        _finish(j0, *bufs[0])

        @pl.when(j0 + 2 < ROWS_PER_W)
        def _next():
            _start(j0 + 2, *bufs[0])

        _finish(j0 + 1, *bufs[1])
        return _
    lax.fori_loop(0, ROWS_PER_W // 2, _pair, 0)

    plsc.subcore_barrier()

    pltpu.sync_copy(acc.at[pl.ds(sid * SLICE, SLICE)],
                    numer_hbm.at[cid, pl.ds(sid * SLICE, SLICE)])
    pltpu.sync_copy(accd.at[pl.ds(sid * SLICE, SLICE)],
                    denom_hbm.at[cid, pl.ds(sid * SLICE, SLICE)])

    @pl.when(sid == 0)
    def _dump_tail():
        pltpu.sync_copy(acc.at[pl.ds(NS * SLICE, TAIL)],
                        numer_hbm.at[cid, pl.ds(NS * SLICE, TAIL)])
        pltpu.sync_copy(accd.at[pl.ds(NS * SLICE, TAIL)],
                        denom_hbm.at[cid, pl.ds(NS * SLICE, TAIL)])


def _sc_edges(feat2, eler, sd):
    mesh = plsc.VectorSubcoreMesh(core_axis_name="c", subcore_axis_name="s")
    chunk_bufs = [
        pltpu.VMEM((CHUNK, HD), jnp.float32),
        pltpu.VMEM((CHUNK, HD), jnp.float32),
        pltpu.VMEM((SLICE,), jnp.float32),
    ]
    per_set = [
        pltpu.VMEM((CHUNK,), jnp.int32),
        pltpu.VMEM((CHUNK,), jnp.int32),
        pltpu.VMEM((CHUNK,), jnp.int32),
        pltpu.VMEM((CHUNK,), jnp.float32),
        pltpu.VMEM((CHUNK,), jnp.float32),
        pltpu.VMEM((CHUNK,), jnp.float32),
    ]
    fn = functools.partial(
        pl.kernel,
        out_type=(
            jax.ShapeDtypeStruct((NC, N, HD), jnp.float32),
            jax.ShapeDtypeStruct((NC, N), jnp.float32),
        ),
        mesh=mesh,
        compiler_params=pltpu.CompilerParams(
            needs_layout_passes=False, use_tc_tiling_on_sc=False),
        scratch_types=[
            pltpu.VMEM((ROWS_PER_W, CHUNK), jnp.int32),
        ] + chunk_bufs + per_set + per_set + [
            pltpu.VMEM_SHARED((N, HD), jnp.float32),
            pltpu.VMEM_SHARED((N + PSRC,), jnp.float32),
            pltpu.VMEM_SHARED((N,), jnp.float32),
            pltpu.VMEM_SHARED((N, HD), jnp.float32),
            pltpu.VMEM_SHARED((N,), jnp.float32),
            pltpu.SemaphoreType.DMA,
            pltpu.SemaphoreType.DMA,
            pltpu.SemaphoreType.DMA,
            pltpu.SemaphoreType.DMA,
            pltpu.SemaphoreType.DMA,
            pltpu.SemaphoreType.DMA,
        ],
    )(_sc_body)
    return fn(feat2, eler, sd)


def _combine_body(n0_ref, n1_ref, d_ref, b_ref, out_ref):
    den = jnp.maximum(d_ref[...], 1e-9)
    out_ref[:, :HD] = n0_ref[...] / den + b_ref[:, :HD]
    out_ref[:, HD:] = n1_ref[...] / den + b_ref[:, HD:]


def _combine(numer, denom, bias):
    n0, n1 = numer[0], numer[1]
    d = denom[0].reshape(N, 1)
    b = bias.reshape(1, D)
    grid = (N // RB,)
    return pl.pallas_call(
        _combine_body,
        grid=grid,
        in_specs=[
            pl.BlockSpec((RB, HD), lambda i: (i, 0)),
            pl.BlockSpec((RB, HD), lambda i: (i, 0)),
            pl.BlockSpec((RB, 1), lambda i: (i, 0)),
            pl.BlockSpec((1, D), lambda i: (0, 0)),
        ],
        out_specs=pl.BlockSpec((RB, D), lambda i: (i, 0)),
        out_shape=jax.ShapeDtypeStruct((N, D), jnp.float32),
    )(n0, n1, d, b)


def kernel(x, edge_index, W, attn_l, attn_r, bias):
    src = edge_index[0]
    dst = edge_index[1]
    pad_src = N + (jnp.arange(NPAD, dtype=jnp.int32) % PSRC)
    pad_dst = (jnp.arange(NPAD, dtype=jnp.int32) * 13) % N
    src_p = jnp.concatenate([src, pad_src])
    dst_p = jnp.concatenate([dst, pad_dst])
    sd = (src_p | (dst_p << 14)).reshape(EPAD // CHUNK, CHUNK)

    feat2, el, er = _project(x, W, attn_l, attn_r)
    eler = jnp.stack([
        jnp.concatenate([el.reshape(N), jnp.full((PSRC,), -1e30, jnp.float32)]),
        jnp.concatenate([er.reshape(N), jnp.zeros((PSRC,), jnp.float32)]),
    ])
    numer, denom = _sc_edges(feat2, eler, sd)
    out = _combine(numer, denom, bias)
    return out.reshape(N, 1, D)


# speedup vs baseline: 1.1259x; 1.0182x over previous
import functools

import jax
import jax.numpy as jnp
from jax import lax
from jax.experimental import pallas as pl
from jax.experimental.pallas import tpu as pltpu
from jax.experimental.pallas import tpu_sc as plsc

N = 10000
E = 320000
D = 128
HD = D // 2

NC = 2
NS = 16
CHUNK = 128
ROWS_PER_W = 160
EPAD = NS * ROWS_PER_W * CHUNK
NPAD = EPAD - E
PSRC = 128
SLICE = 624
TAIL = N - NS * SLICE
RB = 1000


def _project_body(x_ref, w_ref, al_ref, ar_ref, feat_ref, el_ref, er_ref):
    f = jnp.dot(x_ref[...], w_ref[...], preferred_element_type=jnp.float32)
    feat_ref[0] = f[:, :HD]
    feat_ref[1] = f[:, HD:]
    el_ref[...] = jnp.sum(f * al_ref[...], axis=1, keepdims=True)
    er_ref[...] = jnp.sum(f * ar_ref[...], axis=1, keepdims=True)


def _project(x, W, attn_l, attn_r):
    grid = (N // RB,)
    return pl.pallas_call(
        _project_body,
        grid=grid,
        in_specs=[
            pl.BlockSpec((RB, D), lambda i: (i, 0)),
            pl.BlockSpec((D, D), lambda i: (0, 0)),
            pl.BlockSpec((1, D), lambda i: (0, 0)),
            pl.BlockSpec((1, D), lambda i: (0, 0)),
        ],
        out_specs=[
            pl.BlockSpec((NC, RB, HD), lambda i: (0, i, 0)),
            pl.BlockSpec((RB, 1), lambda i: (i, 0)),
            pl.BlockSpec((RB, 1), lambda i: (i, 0)),
        ],
        out_shape=[
            jax.ShapeDtypeStruct((NC, N, HD), jnp.float32),
            jax.ShapeDtypeStruct((N, 1), jnp.float32),
            jax.ShapeDtypeStruct((N, 1), jnp.float32),
        ],
    )(x, W, attn_l, attn_r)


def _sc_body(feat2_hbm, eler_hbm, sd_hbm,
             numer_hbm, denom_hbm,
             sd_v, rows_v, rows_v2, zb_v,
             srcr_v, srcc_v, dstc_v, elc_v, erc_v, eec_v,
             srcr_v2, srcc_v2, dstc_v2, elc_v2, erc_v2, eec_v2,
             feat_sp, el_sp, er_sp, acc, accd,
             sem, seme, sems, sem2, seme2, sems2):
    cid = lax.axis_index("c")
    sid = lax.axis_index("s")
    zero16 = jnp.zeros((16,), jnp.float32)
    nmax16 = jnp.full((16,), N - 1, jnp.int32)

    base = sid * ROWS_PER_W
    pltpu.async_copy(feat2_hbm.at[cid, pl.ds(sid * SLICE, SLICE)],
                     feat_sp.at[pl.ds(sid * SLICE, SLICE)], sem)
    pltpu.async_copy(sd_hbm.at[pl.ds(base, ROWS_PER_W)], sd_v, seme)

    @pl.when(sid == 0)
    def _stage_tail():
        pltpu.async_copy(feat2_hbm.at[cid, pl.ds(NS * SLICE, TAIL)],
                         feat_sp.at[pl.ds(NS * SLICE, TAIL)], sems)
        pltpu.async_copy(eler_hbm.at[0], el_sp, sems)

    @pl.when(sid == 1)
    def _stage_er():
        pltpu.async_copy(eler_hbm.at[1, pl.ds(0, N)], er_sp, sems)

    def _zrow(i, _):
        for c in range(HD // 16):
            rows_v[i, pl.ds(16 * c, 16)] = zero16
        return _
    lax.fori_loop(0, CHUNK, _zrow, 0, unroll=4)

    def _zb(i, _):
        zb_v[pl.ds(16 * i, 16)] = zero16
        return _
    lax.fori_loop(0, SLICE // 16, _zb, 0, unroll=4)

    def _zacc(k, _):
        pltpu.sync_copy(rows_v.at[pl.ds(0, 104)],
                        acc.at[pl.ds(sid * SLICE + k * 104, 104)])
        return _
    lax.fori_loop(0, SLICE // 104, _zacc, 0)
    pltpu.sync_copy(zb_v, accd.at[pl.ds(sid * SLICE, SLICE)])

    @pl.when(sid == 0)
    def _zero_tail():
        pltpu.sync_copy(rows_v.at[pl.ds(0, TAIL)],
                        acc.at[pl.ds(NS * SLICE, TAIL)])
        pltpu.sync_copy(zb_v.at[pl.ds(0, TAIL)],
                        accd.at[pl.ds(NS * SLICE, TAIL)])

    pltpu.make_async_copy(feat2_hbm.at[cid, pl.ds(sid * SLICE, SLICE)],
                          feat_sp.at[pl.ds(sid * SLICE, SLICE)], sem).wait()
    pltpu.make_async_copy(sd_hbm.at[pl.ds(base, ROWS_PER_W)], sd_v,
                          seme).wait()

    @pl.when(sid == 0)
    def _wait_tail():
        pltpu.make_async_copy(feat2_hbm.at[cid, pl.ds(NS * SLICE, TAIL)],
                              feat_sp.at[pl.ds(NS * SLICE, TAIL)],
                              sems).wait()
        pltpu.make_async_copy(eler_hbm.at[0], el_sp, sems).wait()

    @pl.when(sid == 1)
    def _wait_er():
        pltpu.make_async_copy(eler_hbm.at[1, pl.ds(0, N)], er_sp,
                              sems).wait()

    plsc.subcore_barrier()

    mask16 = jnp.full((16,), (1 << 14) - 1, jnp.int32)
    sh14 = jnp.full((16,), 14, jnp.int32)
    lane_splat = [jnp.full((16,), l, jnp.int32) for l in range(16)]

    bufs = ((rows_v, srcr_v, srcc_v, dstc_v, elc_v, erc_v, eec_v,
             sem, seme, sems),
            (rows_v2, srcr_v2, srcc_v2, dstc_v2, elc_v2, erc_v2, eec_v2,
             sem2, seme2, sems2))

    def _start(j, rows, srcr, srcc, dstc, elc, erc, eec, s, se, ss):
        for c in range(CHUNK // 16):
            p16 = sd_v[j, pl.ds(16 * c, 16)]
            s16 = p16 & mask16
            srcr[pl.ds(16 * c, 16)] = s16
            srcc[pl.ds(16 * c, 16)] = jnp.minimum(s16, nmax16)
            dstc[pl.ds(16 * c, 16)] = lax.shift_right_logical(p16, sh14)
        pltpu.async_copy(feat_sp.at[srcc], rows, s)
        pltpu.async_copy(el_sp.at[srcr], elc, se)
        pltpu.async_copy(er_sp.at[dstc], erc, se)

    def _finish(j, rows, srcr, srcc, dstc, elc, erc, eec, s, se, ss):
        pltpu.make_async_copy(el_sp.at[srcr], elc, se).wait()
        pltpu.make_async_copy(er_sp.at[dstc], erc, se).wait()
        pltpu.make_async_copy(feat_sp.at[srcc], rows, s).wait()

        for g in range(CHUNK // 16):
            e = elc[pl.ds(16 * g, 16)] + erc[pl.ds(16 * g, 16)]
            e = jnp.maximum(e, e * 0.2)
            a16 = jnp.exp(e)
            eec[pl.ds(16 * g, 16)] = a16
            for l in range(16):
                al = a16[lane_splat[l]]
                i = 16 * g + l
                for c in range(HD // 16):
                    rows[i, pl.ds(16 * c, 16)] = (
                        rows[i, pl.ds(16 * c, 16)] * al)

        pltpu.sync_copy(rows, acc.at[dstc], add=True)
        pltpu.sync_copy(eec, accd.at[dstc], add=True)

    _start(0, *bufs[0])

    def _pair(jj, _):
        j0 = 2 * jj
        _start(j0 + 1, *bufs[1])
        _finish(j0, *bufs[0])

        @pl.when(j0 + 2 < ROWS_PER_W)
        def _next():
            _start(j0 + 2, *bufs[0])

        _finish(j0 + 1, *bufs[1])
        return _
    lax.fori_loop(0, ROWS_PER_W // 2, _pair, 0)

    plsc.subcore_barrier()

    pltpu.sync_copy(acc.at[pl.ds(sid * SLICE, SLICE)],
                    numer_hbm.at[cid, pl.ds(sid * SLICE, SLICE)])
    pltpu.sync_copy(accd.at[pl.ds(sid * SLICE, SLICE)],
                    denom_hbm.at[cid, pl.ds(sid * SLICE, SLICE)])

    @pl.when(sid == 0)
    def _dump_tail():
        pltpu.sync_copy(acc.at[pl.ds(NS * SLICE, TAIL)],
                        numer_hbm.at[cid, pl.ds(NS * SLICE, TAIL)])
        pltpu.sync_copy(accd.at[pl.ds(NS * SLICE, TAIL)],
                        denom_hbm.at[cid, pl.ds(NS * SLICE, TAIL)])


def _sc_edges(feat2, eler, sd):
    mesh = plsc.VectorSubcoreMesh(core_axis_name="c", subcore_axis_name="s")
    chunk_bufs = [
        pltpu.VMEM((CHUNK, HD), jnp.float32),
        pltpu.VMEM((CHUNK, HD), jnp.float32),
        pltpu.VMEM((SLICE,), jnp.float32),
    ]
    per_set = [
        pltpu.VMEM((CHUNK,), jnp.int32),
        pltpu.VMEM((CHUNK,), jnp.int32),
        pltpu.VMEM((CHUNK,), jnp.int32),
        pltpu.VMEM((CHUNK,), jnp.float32),
        pltpu.VMEM((CHUNK,), jnp.float32),
        pltpu.VMEM((CHUNK,), jnp.float32),
    ]
    fn = functools.partial(
        pl.kernel,
        out_type=(
            jax.ShapeDtypeStruct((NC, N, HD), jnp.float32),
            jax.ShapeDtypeStruct((NC, N), jnp.float32),
        ),
        mesh=mesh,
        compiler_params=pltpu.CompilerParams(
            needs_layout_passes=False, use_tc_tiling_on_sc=False),
        scratch_types=[
            pltpu.VMEM((ROWS_PER_W, CHUNK), jnp.int32),
        ] + chunk_bufs + per_set + per_set + [
            pltpu.VMEM_SHARED((N, HD), jnp.float32),
            pltpu.VMEM_SHARED((N + PSRC,), jnp.float32),
            pltpu.VMEM_SHARED((N,), jnp.float32),
            pltpu.VMEM_SHARED((N, HD), jnp.float32),
            pltpu.VMEM_SHARED((N,), jnp.float32),
            pltpu.SemaphoreType.DMA,
            pltpu.SemaphoreType.DMA,
            pltpu.SemaphoreType.DMA,
            pltpu.SemaphoreType.DMA,
            pltpu.SemaphoreType.DMA,
            pltpu.SemaphoreType.DMA,
        ],
    )(_sc_body)
    return fn(feat2, eler, sd)


def _combine_body(n0_ref, n1_ref, d_ref, b_ref, out_ref):
    den = jnp.maximum(d_ref[...], 1e-9)
    out_ref[:, :HD] = n0_ref[...] / den + b_ref[:, :HD]
    out_ref[:, HD:] = n1_ref[...] / den + b_ref[:, HD:]


def _combine(numer, denom, bias):
    n0, n1 = numer[0], numer[1]
    d = denom[0].reshape(N, 1)
    b = bias.reshape(1, D)
    grid = (N // RB,)
    return pl.pallas_call(
        _combine_body,
        grid=grid,
        in_specs=[
            pl.BlockSpec((RB, HD), lambda i: (i, 0)),
            pl.BlockSpec((RB, HD), lambda i: (i, 0)),
            pl.BlockSpec((RB, 1), lambda i: (i, 0)),
            pl.BlockSpec((1, D), lambda i: (0, 0)),
        ],
        out_specs=pl.BlockSpec((RB, D), lambda i: (i, 0)),
        out_shape=jax.ShapeDtypeStruct((N, D), jnp.float32),
    )(n0, n1, d, b)


def kernel(x, edge_index, W, attn_l, attn_r, bias):
    src = edge_index[0]
    dst = edge_index[1]
    pad_src = N + (jnp.arange(NPAD, dtype=jnp.int32) % PSRC)
    pad_dst = (jnp.arange(NPAD, dtype=jnp.int32) * 13) % N
    src_p = jnp.concatenate([src, pad_src])
    dst_p = jnp.concatenate([dst, pad_dst])
    sd = (src_p | (dst_p << 14)).reshape(EPAD // CHUNK, CHUNK)

    feat2, el, er = _project(x, W, attn_l, attn_r)
    eler = jnp.stack([
        jnp.concatenate([el.reshape(N), jnp.full((PSRC,), -1e30, jnp.float32)]),
        jnp.concatenate([er.reshape(N), jnp.zeros((PSRC,), jnp.float32)]),
    ])
    numer, denom = _sc_edges(feat2, eler, sd)
    out = _combine(numer, denom, bias)
    return out.reshape(N, 1, D)
